# Initial kernel scaffold; baseline (speedup 1.0000x reference)
import jax
import jax.numpy as jnp
from jax.experimental import pallas as pl


def kernel(x, gate_w, w_fc, w_proj):
    raise NotImplementedError("write your pallas kernel here")


# speedup vs baseline: 1.4750x; 1.4750x over previous
import jax
import jax.numpy as jnp
from jax.experimental import pallas as pl
from jax.experimental.pallas import tpu as pltpu

_B, _T, _D, _E, _H, _TOPK = 1, 2048, 768, 8, 1536, 2
_N = _B * _T


def _gate_kernel(x_ref, gw_ref, wd_ref, loss_ref):
    x = x_ref[...]
    gw = gw_ref[...]
    logits = jax.lax.dot_general(
        x, gw, (((1,), (1,)), ((), ())),
        preferred_element_type=jnp.float32)
    p = jax.nn.softmax(logits, axis=-1)
    rank = jnp.zeros_like(p)
    eidx = jax.lax.broadcasted_iota(jnp.int32, (1, _E), 1)
    for j in range(_E):
        pj = p[:, j:j + 1]
        gt = (pj > p).astype(p.dtype)
        eq = ((pj == p) & (j < eidx)).astype(p.dtype)
        rank += gt + eq
    sel = (rank < 2.0).astype(p.dtype)
    w = p * sel
    wd = w / jnp.sum(w, axis=1, keepdims=True)
    wd_ref[...] = wd
    counts = jnp.sum(sel, axis=0)
    pmean = jnp.mean(p, axis=0)
    loss_ref[...] = (jnp.sum(pmean * counts) * (_E / _N)).reshape(1, 1)


def _expert_kernel(wd_ref, x_ref, wfc_ref, wproj_ref, out_ref):
    e = pl.program_id(1)
    x = x_ref[...].astype(jnp.bfloat16)
    wfc = wfc_ref[0].astype(jnp.bfloat16)
    h = jax.lax.dot_general(
        x, wfc, (((1,), (1,)), ((), ())),
        preferred_element_type=jnp.float32)
    h = jnp.square(jnp.maximum(h, 0.0)).astype(jnp.bfloat16)
    wp = wproj_ref[0].astype(jnp.bfloat16)
    y = jax.lax.dot_general(
        h, wp, (((1,), (1,)), ((), ())),
        preferred_element_type=jnp.float32)
    eidx = jax.lax.broadcasted_iota(jnp.int32, (1, _E), 1)
    wcol = jnp.sum(wd_ref[...] * (eidx == e), axis=1, keepdims=True)
    contrib = wcol * y

    @pl.when(e == 0)
    def _():
        out_ref[...] = contrib

    @pl.when(e != 0)
    def _():
        out_ref[...] += contrib


def kernel(x, gate_w, w_fc, w_proj):
    x_flat = x.reshape(_N, _D)
    wd, loss = pl.pallas_call(
        _gate_kernel,
        out_shape=[
            jax.ShapeDtypeStruct((_N, _E), jnp.float32),
            jax.ShapeDtypeStruct((1, 1), jnp.float32),
        ],
    )(x_flat, gate_w)

    TN = 1024
    n_t = _N // TN
    out = pl.pallas_call(
        _expert_kernel,
        grid=(n_t, _E),
        in_specs=[
            pl.BlockSpec((TN, _E), lambda t, e: (t, 0)),
            pl.BlockSpec((TN, _D), lambda t, e: (t, 0)),
            pl.BlockSpec((1, _H, _D), lambda t, e: (e, 0, 0)),
            pl.BlockSpec((1, _D, _H), lambda t, e: (e, 0, 0)),
        ],
        out_specs=pl.BlockSpec((TN, _D), lambda t, e: (t, 0)),
        out_shape=jax.ShapeDtypeStruct((_N, _D), jnp.float32),
    )(wd, x_flat, w_fc, w_proj)

    return out.reshape(_B, _T, _D), loss[0, 0]


# speedup vs baseline: 1.4753x; 1.0002x over previous
import jax
import jax.numpy as jnp
from jax.experimental import pallas as pl
from jax.experimental.pallas import tpu as pltpu

_B, _T, _D, _E, _H, _TOPK = 1, 2048, 768, 8, 1536, 2
_N = _B * _T


def _gate_kernel(x_ref, gw_ref, wd_ref, loss_ref):
    x = x_ref[...]
    gw = gw_ref[...]
    logits = jax.lax.dot_general(
        x, gw, (((1,), (1,)), ((), ())),
        preferred_element_type=jnp.float32)
    p = jax.nn.softmax(logits, axis=-1)
    rank = jnp.zeros_like(p)
    eidx = jax.lax.broadcasted_iota(jnp.int32, (1, _E), 1)
    for j in range(_E):
        pj = p[:, j:j + 1]
        gt = (pj > p).astype(p.dtype)
        eq = ((pj == p) & (j < eidx)).astype(p.dtype)
        rank += gt + eq
    sel = (rank < 2.0).astype(p.dtype)
    w = p * sel
    wd = w / jnp.sum(w, axis=1, keepdims=True)
    wd_ref[...] = wd
    counts = jnp.sum(sel, axis=0)
    pmean = jnp.mean(p, axis=0)
    loss_ref[...] = (jnp.sum(pmean * counts) * (_E / _N)).reshape(1, 1)


def _expert_kernel(wd_ref, x_ref, wfc_ref, wproj_ref, out_ref):
    e = pl.program_id(1)
    x = x_ref[...].astype(jnp.bfloat16)
    wfc = wfc_ref[0].astype(jnp.bfloat16)
    h = jax.lax.dot_general(
        x, wfc, (((1,), (1,)), ((), ())),
        preferred_element_type=jnp.float32)
    h = jnp.square(jnp.maximum(h, 0.0)).astype(jnp.bfloat16)
    wp = wproj_ref[0].astype(jnp.bfloat16)
    y = jax.lax.dot_general(
        h, wp, (((1,), (1,)), ((), ())),
        preferred_element_type=jnp.float32)
    eidx = jax.lax.broadcasted_iota(jnp.int32, (1, _E), 1)
    wcol = jnp.sum(wd_ref[...] * (eidx == e), axis=1, keepdims=True)
    contrib = wcol * y

    @pl.when(e == 0)
    def _():
        out_ref[...] = contrib

    @pl.when(e != 0)
    def _():
        out_ref[...] += contrib


def kernel(x, gate_w, w_fc, w_proj):
    x_flat = x.reshape(_N, _D)
    wd, loss = pl.pallas_call(
        _gate_kernel,
        out_shape=[
            jax.ShapeDtypeStruct((_N, _E), jnp.float32),
            jax.ShapeDtypeStruct((1, 1), jnp.float32),
        ],
    )(x_flat, gate_w)

    TN = 1024
    n_t = _N // TN
    out = pl.pallas_call(
        _expert_kernel,
        grid=(n_t, _E),
        in_specs=[
            pl.BlockSpec((TN, _E), lambda t, e: (t, 0)),
            pl.BlockSpec((TN, _D), lambda t, e: (t, 0)),
            pl.BlockSpec((1, _H, _D), lambda t, e: (e, 0, 0)),
            pl.BlockSpec((1, _D, _H), lambda t, e: (e, 0, 0)),
        ],
        out_specs=pl.BlockSpec((TN, _D), lambda t, e: (t, 0)),
        out_shape=jax.ShapeDtypeStruct((_N, _D), jnp.float32),
        compiler_params=pltpu.CompilerParams(
            dimension_semantics=("parallel", "arbitrary")),
    )(wd, x_flat, w_fc, w_proj)

    return out.reshape(_B, _T, _D), loss[0, 0]
